# baseline (speedup 1.0000x reference)
import jax
import jax.numpy as jnp
from jax import lax
from jax.experimental import pallas as pl
from jax.experimental.pallas import tpu as pltpu
from jax.experimental.pallas import tpu_sc as plsc

D_MODEL = 64
LANES = 16
W = 512
SCALE = 8.0


def _gather_scale(table_hbm, i_hbm, o_hbm):
    def body(i_vmem, o_vmem):
        pltpu.sync_copy(table_hbm.at[i_vmem.at[0]], o_vmem)
        iv_ref = i_vmem.at[0]

        @pl.loop(0, W, step=LANES)
        def _(g):
            iv = iv_ref[pl.ds(g, LANES)]
            fv = jnp.where(iv != 0, SCALE, 0.0).astype(jnp.float32)
            for j in range(LANES):
                fj = lax.gather(
                    fv, jnp.full((LANES, 1), j, jnp.int32),
                    dimension_numbers=lax.GatherDimensionNumbers(
                        offset_dims=(), collapsed_slice_dims=(0,),
                        start_index_map=(0,)),
                    slice_sizes=(1,),
                    mode=lax.GatherScatterMode.PROMISE_IN_BOUNDS)
                row_ref = o_vmem.at[g + j]
                for c in range(0, D_MODEL, LANES):
                    row_ref[pl.ds(c, LANES)] = row_ref[pl.ds(c, LANES)] * fj

    return body


def kernel(x, table):
    b, s = x.shape
    n = b * s
    idx = x.reshape(1, n)
    mesh = plsc.VectorSubcoreMesh(core_axis_name="core",
                                  subcore_axis_name="subcore")

    @pl.kernel(out_type=jax.ShapeDtypeStruct((n, D_MODEL), jnp.float32),
               mesh=mesh,
               compiler_params=pltpu.CompilerParams(use_tc_tiling_on_sc=False))
    def run(table_hbm, i_hbm, o_hbm):
        pltpu.emit_pipeline(
            _gather_scale(table_hbm, i_hbm, o_hbm),
            grid=(n // W,),
            in_specs=[pl.BlockSpec((1, W), lambda i: (0, i))],
            out_specs=[pl.BlockSpec((W, D_MODEL), lambda i: (i, 0))],
            core_axis_name=("core", "subcore"),
            dimension_semantics=(pltpu.PARALLEL,),
        )(i_hbm, o_hbm)

    out = run(table, idx)
    return out.reshape(b, s, D_MODEL)
